# Initial kernel scaffold; baseline (speedup 1.0000x reference)
import jax
import jax.numpy as jnp
from jax.experimental import pallas as pl


def kernel(x, params):
    raise NotImplementedError("write your pallas kernel here")


# speedup vs baseline: 1.1710x; 1.1710x over previous
import jax
import jax.numpy as jnp
from jax.experimental import pallas as pl


def _mm_kernel(w_ref, x_ref, o_ref):
    o_ref[...] = jnp.dot(w_ref[...], x_ref[...],
                         preferred_element_type=jnp.float32)


def _pmatmul(w, x2d):
    M, K = w.shape
    N = x2d.shape[1]
    if K % 8:
        pad = 8 - K % 8
        w = jnp.pad(w, ((0, 0), (0, pad)))
        x2d = jnp.pad(x2d, ((0, pad), (0, 0)))
        K += pad
    bm = 256 if M % 256 == 0 else M
    bn = 2048 if N % 2048 == 0 else 512
    if N % bn:
        bn = N
    grid = (M // bm, N // bn)
    return pl.pallas_call(
        _mm_kernel,
        grid=grid,
        in_specs=[pl.BlockSpec((bm, K), lambda i, j: (i, 0)),
                  pl.BlockSpec((K, bn), lambda i, j: (0, j))],
        out_specs=pl.BlockSpec((bm, bn), lambda i, j: (i, j)),
        out_shape=jax.ShapeDtypeStruct((M, N), jnp.float32),
    )(w, x2d)


def _conv1d(x, w, b=None):
    B, C, N = x.shape
    x2d = x.transpose(1, 0, 2).reshape(C, B * N)
    y2d = _pmatmul(w, x2d)
    y = y2d.reshape(w.shape[0], B, N).transpose(1, 0, 2)
    if b is not None:
        y = y + b[None, :, None]
    return y


def _lowatt_kernel(q_ref, k_ref, v_ref, att_ref, xr_ref):
    q = q_ref[0]
    k = k_ref[0]
    v = v_ref[0]
    e = jnp.dot(q, k, preferred_element_type=jnp.float32)
    e = e - jnp.max(e, axis=-1, keepdims=True)
    a = jnp.exp(e)
    a = a / jnp.sum(a, axis=-1, keepdims=True)
    a = a / (1e-9 + jnp.sum(a, axis=0, keepdims=True))
    att_ref[0] = a
    xr_ref[0] = jnp.dot(v, a, preferred_element_type=jnp.float32)


def _low_attention_core(xq, xk, xv):
    B, Np, C4 = xq.shape
    C = xv.shape[1]
    return pl.pallas_call(
        _lowatt_kernel,
        grid=(B,),
        in_specs=[pl.BlockSpec((1, Np, C4), lambda i: (i, 0, 0)),
                  pl.BlockSpec((1, C4, Np), lambda i: (i, 0, 0)),
                  pl.BlockSpec((1, C, Np), lambda i: (i, 0, 0))],
        out_specs=[pl.BlockSpec((1, Np, Np), lambda i: (i, 0, 0)),
                   pl.BlockSpec((1, C, Np), lambda i: (i, 0, 0))],
        out_shape=[jax.ShapeDtypeStruct((B, Np, Np), jnp.float32),
                   jax.ShapeDtypeStruct((B, C, Np), jnp.float32)],
    )(xq, xk, xv)


def _bmm_kernel(a_ref, b_ref, o_ref):
    o_ref[0] = jnp.dot(a_ref[0], b_ref[0],
                       preferred_element_type=jnp.float32)


def _pbmm(a, b):
    B, M, K = a.shape
    N = b.shape[2]
    return pl.pallas_call(
        _bmm_kernel,
        grid=(B,),
        in_specs=[pl.BlockSpec((1, M, K), lambda i: (i, 0, 0)),
                  pl.BlockSpec((1, K, N), lambda i: (i, 0, 0))],
        out_specs=pl.BlockSpec((1, M, N), lambda i: (i, 0, 0)),
        out_shape=jax.ShapeDtypeStruct((B, M, N), jnp.float32),
    )(a, b)


def _batchnorm(x, g, b, eps=1e-5):
    m = jnp.mean(x, axis=(0, 2), keepdims=True)
    v = jnp.var(x, axis=(0, 2), keepdims=True)
    return ((x - m) / jnp.sqrt(v + eps)) * g[None, :, None] + b[None, :, None]


def _relu(x):
    return jnp.maximum(x, 0.0)


def _index_points(points, idx):
    B = points.shape[0]
    batch = jnp.arange(B).reshape((B,) + (1,) * (idx.ndim - 1))
    return points[batch, idx]


def _square_distance(src, dst):
    return (jnp.sum(src ** 2, -1)[:, :, None]
            + jnp.sum(dst ** 2, -1)[:, None, :]
            - 2.0 * jnp.matmul(src, dst.transpose(0, 2, 1)))


def _farthest_point_sample(xyz, npoint):
    xyz = jax.lax.stop_gradient(xyz)
    B, N, _ = xyz.shape

    def body(i, state):
        idxs, dist, far = state
        idxs = idxs.at[:, i].set(far)
        centroid = jnp.take_along_axis(xyz, far[:, None, None], axis=1)
        d = jnp.sum((xyz - centroid) ** 2, -1)
        dist = jnp.minimum(dist, d)
        far = jnp.argmax(dist, -1).astype(jnp.int32)
        return idxs, dist, far

    idxs = jnp.zeros((B, npoint), jnp.int32)
    dist = jnp.full((B, N), 1e10, jnp.float32)
    far = jnp.zeros((B,), jnp.int32)
    idxs, _, _ = jax.lax.fori_loop(0, npoint, body, (idxs, dist, far))
    return idxs


def _knn_point(k, xyz, new_xyz):
    sqr = _square_distance(jax.lax.stop_gradient(new_xyz),
                           jax.lax.stop_gradient(xyz))
    _, idx = jax.lax.top_k(-sqr, k)
    return idx


def _sample_and_group(npoint, nsample, xyz, points):
    fps_idx = _farthest_point_sample(xyz, npoint)
    new_xyz = _index_points(xyz, fps_idx)
    new_points = _index_points(points, fps_idx)
    idx = _knn_point(nsample, xyz, new_xyz)
    grouped_points = _index_points(points, idx)
    grouped_norm = grouped_points - new_points[:, :, None, :]
    new_feat = jnp.concatenate(
        [grouped_norm,
         jnp.broadcast_to(new_points[:, :, None, :], grouped_norm.shape)], -1)
    return new_xyz, new_feat


def _sg_forward(p, s, x, coords):
    xt = x.transpose(0, 2, 1)
    new_xyz, nf = _sample_and_group(s, 32, coords, xt)
    b, s_, k_, d = nf.shape
    nf = nf.transpose(0, 1, 3, 2).reshape(b * s_, d, k_)
    nf = _relu(_batchnorm(_conv1d(nf, p['w1']), p['g1'], p['b1']))
    nf = _relu(_batchnorm(_conv1d(nf, p['w2']), p['g2'], p['b2']))
    nf = jnp.max(nf, -1)
    nf = nf.reshape(b, s_, -1).transpose(0, 2, 1)
    return new_xyz, nf


def _neighbor_embedding(p, x, samples):
    xyz = x.transpose(0, 2, 1)
    f = _relu(_batchnorm(_conv1d(x, p['w1']), p['g1'], p['b1']))
    f = _relu(_batchnorm(_conv1d(f, p['w2']), p['g2'], p['b2']))
    xyz1, f1 = _sg_forward(p['sg1'], samples[0], f, xyz)
    xyz2, f2 = _sg_forward(p['sg2'], samples[1], f1, xyz1)
    return xyz2, f2


def _low_attention_f(p, x):
    xq = _conv1d(x, p['wq']).transpose(0, 2, 1)
    xk = _conv1d(x, p['wq'])
    xv = _conv1d(x, p['wv'], p['bv'])
    att, xr = _low_attention_core(xq, xk, xv)
    xr = _relu(_batchnorm(_conv1d(x - xr, p['wt'], p['bt']), p['g'], p['b']))
    return x + xr, att


def _full_attention_f(p, x, att, fps_idx, knn_idx):
    B, C, N = x.shape
    xv = _conv1d(x, p['wv'], p['bv'])
    low_v = _index_points(xv.transpose(0, 2, 1), fps_idx).transpose(0, 2, 1)
    low_r = _pbmm(low_v, att).transpose(0, 2, 1)
    S, K = knn_idx.shape[1], knn_idx.shape[2]
    vals = jnp.broadcast_to(low_r[:, :, None, :], (B, S, K, C))
    batch = jnp.arange(B)[:, None, None]
    up = jnp.zeros((B, N, C), x.dtype).at[batch, knn_idx].add(vals)
    cnt = jnp.zeros((B, N, 1), x.dtype).at[batch, knn_idx].add(
        jnp.ones((B, S, K, 1), x.dtype))
    xr = (up / jnp.maximum(cnt, 1.0)).transpose(0, 2, 1)
    xr = _relu(_batchnorm(_conv1d(x - xr, p['wt'], p['bt']), p['g'], p['b']))
    return x + xr


def kernel(x, params):
    samples = (512, 256)
    coords, feat = _neighbor_embedding(params['ne'], x, samples)
    fps_idx = _farthest_point_sample(coords, 64)
    low_coords = _index_points(coords, fps_idx)
    low_x = _index_points(feat.transpose(0, 2, 1), fps_idx).transpose(0, 2, 1)
    knn_idx = _knn_point(16, coords, low_coords)
    lx1, att = _low_attention_f(params['la1'], low_x)
    x1 = _full_attention_f(params['ha1'], feat, att, fps_idx, knn_idx)
    lx2, att = _low_attention_f(params['la2'], lx1)
    x2 = _full_attention_f(params['ha2'], x1, att, fps_idx, knn_idx)
    lx3, att = _low_attention_f(params['la3'], lx2)
    x3 = _full_attention_f(params['ha3'], x2, att, fps_idx, knn_idx)
    _, att = _low_attention_f(params['la4'], lx3)
    x4 = _full_attention_f(params['ha4'], x3, att, fps_idx, knn_idx)
    xc = jnp.concatenate([feat, x1, x2, x3, x4], 1)
    xl = _batchnorm(_conv1d(xc, params['lin_w']), params['lin_g'],
                    params['lin_b'])
    xl = jnp.where(xl >= 0, xl, 0.2 * xl)
    x_max = jnp.max(xl, -1)
    x_mean = jnp.mean(xl, -1)
    return xl, x_max, x_mean


# speedup vs baseline: 1.2495x; 1.0670x over previous
import functools

import jax
import jax.numpy as jnp
from jax.experimental import pallas as pl


def _mm_kernel(w_ref, x_ref, o_ref):
    o_ref[...] = jnp.dot(w_ref[...], x_ref[...],
                         preferred_element_type=jnp.float32)


def _pmatmul(w, x2d):
    M, K = w.shape
    N = x2d.shape[1]
    if K % 8:
        pad = 8 - K % 8
        w = jnp.pad(w, ((0, 0), (0, pad)))
        x2d = jnp.pad(x2d, ((0, pad), (0, 0)))
        K += pad
    bm = 256 if M % 256 == 0 else M
    bn = 2048 if N % 2048 == 0 else 512
    if N % bn:
        bn = N
    grid = (M // bm, N // bn)
    return pl.pallas_call(
        _mm_kernel,
        grid=grid,
        in_specs=[pl.BlockSpec((bm, K), lambda i, j: (i, 0)),
                  pl.BlockSpec((K, bn), lambda i, j: (0, j))],
        out_specs=pl.BlockSpec((bm, bn), lambda i, j: (i, j)),
        out_shape=jax.ShapeDtypeStruct((M, N), jnp.float32),
    )(w, x2d)


def _conv1d(x, w, b=None):
    B, C, N = x.shape
    x2d = x.transpose(1, 0, 2).reshape(C, B * N)
    y2d = _pmatmul(w, x2d)
    y = y2d.reshape(w.shape[0], B, N).transpose(1, 0, 2)
    if b is not None:
        y = y + b[None, :, None]
    return y


def _lowatt_kernel(q_ref, k_ref, v_ref, att_ref, xr_ref):
    q = q_ref[0]
    k = k_ref[0]
    v = v_ref[0]
    e = jnp.dot(q, k, preferred_element_type=jnp.float32)
    e = e - jnp.max(e, axis=-1, keepdims=True)
    a = jnp.exp(e)
    a = a / jnp.sum(a, axis=-1, keepdims=True)
    a = a / (1e-9 + jnp.sum(a, axis=0, keepdims=True))
    att_ref[0] = a
    xr_ref[0] = jnp.dot(v, a, preferred_element_type=jnp.float32)


def _low_attention_core(xq, xk, xv):
    B, Np, C4 = xq.shape
    C = xv.shape[1]
    return pl.pallas_call(
        _lowatt_kernel,
        grid=(B,),
        in_specs=[pl.BlockSpec((1, Np, C4), lambda i: (i, 0, 0)),
                  pl.BlockSpec((1, C4, Np), lambda i: (i, 0, 0)),
                  pl.BlockSpec((1, C, Np), lambda i: (i, 0, 0))],
        out_specs=[pl.BlockSpec((1, Np, Np), lambda i: (i, 0, 0)),
                   pl.BlockSpec((1, C, Np), lambda i: (i, 0, 0))],
        out_shape=[jax.ShapeDtypeStruct((B, Np, Np), jnp.float32),
                   jax.ShapeDtypeStruct((B, C, Np), jnp.float32)],
    )(xq, xk, xv)


def _bmm_kernel(a_ref, b_ref, o_ref):
    o_ref[0] = jnp.dot(a_ref[0], b_ref[0],
                       preferred_element_type=jnp.float32)


def _pbmm(a, b):
    B, M, K = a.shape
    N = b.shape[2]
    return pl.pallas_call(
        _bmm_kernel,
        grid=(B,),
        in_specs=[pl.BlockSpec((1, M, K), lambda i: (i, 0, 0)),
                  pl.BlockSpec((1, K, N), lambda i: (i, 0, 0))],
        out_specs=pl.BlockSpec((1, M, N), lambda i: (i, 0, 0)),
        out_shape=jax.ShapeDtypeStruct((B, M, N), jnp.float32),
    )(a, b)


def _batchnorm(x, g, b, eps=1e-5):
    m = jnp.mean(x, axis=(0, 2), keepdims=True)
    v = jnp.var(x, axis=(0, 2), keepdims=True)
    return ((x - m) / jnp.sqrt(v + eps)) * g[None, :, None] + b[None, :, None]


def _relu(x):
    return jnp.maximum(x, 0.0)


def _index_points(points, idx):
    B = points.shape[0]
    batch = jnp.arange(B).reshape((B,) + (1,) * (idx.ndim - 1))
    return points[batch, idx]


def _square_distance(src, dst):
    return (jnp.sum(src ** 2, -1)[:, :, None]
            + jnp.sum(dst ** 2, -1)[:, None, :]
            - 2.0 * jnp.matmul(src, dst.transpose(0, 2, 1)))


def _fps_kernel(npoint, N, xs_ref, ys_ref, zs_ref, out_ref):
    x = xs_ref[0]
    y = ys_ref[0]
    z = zs_ref[0]
    R = x.shape[0]
    NP_R = out_ref.shape[1]
    flat_idx = (jax.lax.broadcasted_iota(jnp.int32, (R, 128), 0) * 128
                + jax.lax.broadcasted_iota(jnp.int32, (R, 128), 1))
    pos_idx = (jax.lax.broadcasted_iota(jnp.int32, (NP_R, 128), 0) * 128
               + jax.lax.broadcasted_iota(jnp.int32, (NP_R, 128), 1))

    def body(i, carry):
        dist, far, idxs = carry
        idxs = jnp.where(pos_idx == i, far, idxs)
        sel = flat_idx == far
        cx = jnp.sum(jnp.where(sel, x, 0.0))
        cy = jnp.sum(jnp.where(sel, y, 0.0))
        cz = jnp.sum(jnp.where(sel, z, 0.0))
        dx = x - cx
        dy = y - cy
        dz = z - cz
        d = dx * dx + dy * dy + dz * dz
        dist = jnp.minimum(dist, d)
        m = jnp.max(dist)
        far = jnp.min(jnp.where(dist == m, flat_idx, N)).astype(jnp.int32)
        return dist, far, idxs

    dist0 = jnp.full((R, 128), 1e10, jnp.float32)
    idxs0 = jnp.zeros((NP_R, 128), jnp.int32)
    _, _, idxs = jax.lax.fori_loop(0, npoint, body,
                                   (dist0, jnp.int32(0), idxs0))
    out_ref[0] = idxs


def _farthest_point_sample(xyz, npoint):
    xyz = jax.lax.stop_gradient(xyz)
    B, N, _ = xyz.shape
    R = N // 128
    xs = xyz[:, :, 0].reshape(B, R, 128)
    ys = xyz[:, :, 1].reshape(B, R, 128)
    zs = xyz[:, :, 2].reshape(B, R, 128)
    np_r = -(-npoint // 128)
    idxs = pl.pallas_call(
        functools.partial(_fps_kernel, npoint, N),
        grid=(B,),
        in_specs=[pl.BlockSpec((1, R, 128), lambda i: (i, 0, 0)),
                  pl.BlockSpec((1, R, 128), lambda i: (i, 0, 0)),
                  pl.BlockSpec((1, R, 128), lambda i: (i, 0, 0))],
        out_specs=pl.BlockSpec((1, np_r, 128), lambda i: (i, 0, 0)),
        out_shape=jax.ShapeDtypeStruct((B, np_r, 128), jnp.int32),
    )(xs, ys, zs)
    return idxs.reshape(B, np_r * 128)[:, :npoint]


def _knn_point(k, xyz, new_xyz):
    sqr = _square_distance(jax.lax.stop_gradient(new_xyz),
                           jax.lax.stop_gradient(xyz))
    _, idx = jax.lax.top_k(-sqr, k)
    return idx


def _sample_and_group(npoint, nsample, xyz, points):
    fps_idx = _farthest_point_sample(xyz, npoint)
    new_xyz = _index_points(xyz, fps_idx)
    new_points = _index_points(points, fps_idx)
    idx = _knn_point(nsample, xyz, new_xyz)
    grouped_points = _index_points(points, idx)
    grouped_norm = grouped_points - new_points[:, :, None, :]
    new_feat = jnp.concatenate(
        [grouped_norm,
         jnp.broadcast_to(new_points[:, :, None, :], grouped_norm.shape)], -1)
    return new_xyz, new_feat


def _sg_forward(p, s, x, coords):
    xt = x.transpose(0, 2, 1)
    new_xyz, nf = _sample_and_group(s, 32, coords, xt)
    b, s_, k_, d = nf.shape
    nf = nf.transpose(0, 1, 3, 2).reshape(b * s_, d, k_)
    nf = _relu(_batchnorm(_conv1d(nf, p['w1']), p['g1'], p['b1']))
    nf = _relu(_batchnorm(_conv1d(nf, p['w2']), p['g2'], p['b2']))
    nf = jnp.max(nf, -1)
    nf = nf.reshape(b, s_, -1).transpose(0, 2, 1)
    return new_xyz, nf


def _neighbor_embedding(p, x, samples):
    xyz = x.transpose(0, 2, 1)
    f = _relu(_batchnorm(_conv1d(x, p['w1']), p['g1'], p['b1']))
    f = _relu(_batchnorm(_conv1d(f, p['w2']), p['g2'], p['b2']))
    xyz1, f1 = _sg_forward(p['sg1'], samples[0], f, xyz)
    xyz2, f2 = _sg_forward(p['sg2'], samples[1], f1, xyz1)
    return xyz2, f2


def _low_attention_f(p, x):
    xq = _conv1d(x, p['wq']).transpose(0, 2, 1)
    xk = _conv1d(x, p['wq'])
    xv = _conv1d(x, p['wv'], p['bv'])
    att, xr = _low_attention_core(xq, xk, xv)
    xr = _relu(_batchnorm(_conv1d(x - xr, p['wt'], p['bt']), p['g'], p['b']))
    return x + xr, att


def _full_attention_f(p, x, att, fps_idx, knn_idx):
    B, C, N = x.shape
    xv = _conv1d(x, p['wv'], p['bv'])
    low_v = _index_points(xv.transpose(0, 2, 1), fps_idx).transpose(0, 2, 1)
    low_r = _pbmm(low_v, att).transpose(0, 2, 1)
    S, K = knn_idx.shape[1], knn_idx.shape[2]
    vals = jnp.broadcast_to(low_r[:, :, None, :], (B, S, K, C))
    batch = jnp.arange(B)[:, None, None]
    up = jnp.zeros((B, N, C), x.dtype).at[batch, knn_idx].add(vals)
    cnt = jnp.zeros((B, N, 1), x.dtype).at[batch, knn_idx].add(
        jnp.ones((B, S, K, 1), x.dtype))
    xr = (up / jnp.maximum(cnt, 1.0)).transpose(0, 2, 1)
    xr = _relu(_batchnorm(_conv1d(x - xr, p['wt'], p['bt']), p['g'], p['b']))
    return x + xr


def kernel(x, params):
    samples = (512, 256)
    coords, feat = _neighbor_embedding(params['ne'], x, samples)
    fps_idx = _farthest_point_sample(coords, 64)
    low_coords = _index_points(coords, fps_idx)
    low_x = _index_points(feat.transpose(0, 2, 1), fps_idx).transpose(0, 2, 1)
    knn_idx = _knn_point(16, coords, low_coords)
    lx1, att = _low_attention_f(params['la1'], low_x)
    x1 = _full_attention_f(params['ha1'], feat, att, fps_idx, knn_idx)
    lx2, att = _low_attention_f(params['la2'], lx1)
    x2 = _full_attention_f(params['ha2'], x1, att, fps_idx, knn_idx)
    lx3, att = _low_attention_f(params['la3'], lx2)
    x3 = _full_attention_f(params['ha3'], x2, att, fps_idx, knn_idx)
    _, att = _low_attention_f(params['la4'], lx3)
    x4 = _full_attention_f(params['ha4'], x3, att, fps_idx, knn_idx)
    xc = jnp.concatenate([feat, x1, x2, x3, x4], 1)
    xl = _batchnorm(_conv1d(xc, params['lin_w']), params['lin_g'],
                    params['lin_b'])
    xl = jnp.where(xl >= 0, xl, 0.2 * xl)
    x_max = jnp.max(xl, -1)
    x_mean = jnp.mean(xl, -1)
    return xl, x_max, x_mean
